# Initial kernel scaffold; baseline (speedup 1.0000x reference)
import jax
import jax.numpy as jnp
from jax.experimental import pallas as pl


def kernel(species, y, W_big, b_big, W_small, b_small, self_energies):
    raise NotImplementedError("write your pallas kernel here")


# speedup vs baseline: 3.2352x; 3.2352x over previous
import jax
import jax.numpy as jnp
from jax.experimental import pallas as pl

_BM = 32


def _tc_body(s_ref, y_ref, w_ref, c_ref, o_ref):
    y = y_ref[...]
    s = s_ref[...]
    w = w_ref[...]
    c = c_ref[...]
    kf = y.shape[-1]
    parts = []
    for k in range(4):
        wk = w[k:k + 1, :].reshape(1, 1, kf)
        parts.append(jnp.sum(y * wk, axis=2, keepdims=True))
    e01 = jnp.where(s == 1, parts[1], parts[0])
    e23 = jnp.where(s == 3, parts[3], parts[2])
    e = jnp.where(s < 2, e01, e23)
    ck = [c[k:k + 1, 0:1].reshape(1, 1, 1) for k in range(4)]
    c01 = jnp.where(s == 1, ck[1], ck[0])
    c23 = jnp.where(s == 3, ck[3], ck[2])
    ca = jnp.where(s < 2, c01, c23)
    o_ref[...] = jnp.sum(e + ca, axis=1)


def kernel(species, y, W_big, b_big, W_small, b_small, self_energies):
    b, a, nn, f = y.shape
    fs = W_small.shape[-1]
    kf = nn * f
    wb = jnp.transpose(W_big, (1, 0, 2))
    ws = jnp.pad(jnp.transpose(W_small, (1, 0, 2)),
                 ((0, 0), (0, 0), (0, f - fs)))
    w_tab = (jnp.concatenate([wb, ws], axis=0).reshape(4, kf)
             * (1.0 / nn)).astype(jnp.float32)
    c_tab = (jnp.concatenate([jnp.sum(b_big, 0), jnp.sum(b_small, 0)], 0) / nn
             + self_energies).reshape(4, 1).astype(jnp.float32)

    y3 = y.reshape(b, a, kf)
    s3 = species.reshape(b, a, 1).astype(jnp.int32)

    out = pl.pallas_call(
        _tc_body,
        grid=(b // _BM,),
        in_specs=[
            pl.BlockSpec((_BM, a, 1), lambda i: (i, 0, 0)),
            pl.BlockSpec((_BM, a, kf), lambda i: (i, 0, 0)),
            pl.BlockSpec((4, kf), lambda i: (0, 0)),
            pl.BlockSpec((4, 1), lambda i: (0, 0)),
        ],
        out_specs=pl.BlockSpec((_BM, 1), lambda i: (i, 0)),
        out_shape=jax.ShapeDtypeStruct((b, 1), jnp.float32),
    )(s3, y3, w_tab, c_tab)
    return out.reshape(b)
